# baseline (speedup 1.0000x reference)
import functools
import math

import jax
import jax.numpy as jnp
from jax import lax
from jax.experimental import pallas as pl
from jax.experimental.pallas import tpu as pltpu
from jax.experimental.pallas import tpu_sc as plsc

_K = 2


def _cumsum_excl(x, T):
    acc = x
    k = 1
    while k < T:
        shifted = jnp.concatenate(
            [jnp.zeros((k,) + x.shape[1:], x.dtype), acc[:-k]], axis=0)
        acc = acc + shifted
        k *= 2
    return acc - x


def _routing_body(cap, h_ref, wg_ref, ids_ref, wk_ref, rank_ref, sel_ref):
    h = h_ref[...]
    wg = wg_ref[...]
    T = h.shape[0]
    E = wg.shape[1]
    logits = jnp.dot(h.astype(jnp.bfloat16), wg.astype(jnp.bfloat16),
                     preferred_element_type=jnp.float32)
    lane = lax.broadcasted_iota(jnp.int32, (T, E), 1)

    m = jnp.max(logits, axis=-1, keepdims=True)
    p = jnp.exp(logits - m)
    s = jnp.sum(p, axis=-1, keepdims=True)
    probs = p / s

    m1 = jnp.max(probs, axis=-1, keepdims=True)
    i1 = jnp.min(jnp.where(probs == m1, lane, E), axis=-1, keepdims=True)
    masked = jnp.where(lane == i1, -jnp.inf, probs)
    m2 = jnp.max(masked, axis=-1, keepdims=True)
    i2 = jnp.min(jnp.where(masked == m2, lane, E), axis=-1, keepdims=True)
    denom = m1 + m2
    w1 = m1 / denom
    w2 = m2 / denom
    ids_ref[...] = jnp.concatenate([i1, i2], axis=1)

    w_full = jnp.where(lane == i1, w1, 0.0) + jnp.where(lane == i2, w2, 0.0)

    keys = lax.bitcast_convert_type(w_full, jnp.int32)
    lo = jnp.zeros((1, E), jnp.int32)
    for b in range(30, -1, -1):
        trial = lo | (1 << b)
        cnt = jnp.sum((keys >= trial).astype(jnp.int32), axis=0, keepdims=True)
        lo = jnp.where(cnt >= cap, trial, lo)
    tau = lo
    g = jnp.sum((keys > tau).astype(jnp.int32), axis=0, keepdims=True)
    tie = (keys == tau)
    tie_rank = _cumsum_excl(tie.astype(jnp.int32), T)
    keep = (keys > tau) | (tie & (tie_rank < (cap - g)))
    wk = jnp.where(keep, w_full, 0.0)
    wk_ref[...] = wk

    pos = (wk > 0).astype(jnp.int32)
    rank = _cumsum_excl(pos, T)
    rank_f = rank.astype(jnp.float32)
    rank_ref[...] = rank_f

    slot = lax.broadcasted_iota(jnp.int32, (T, cap), 1).astype(jnp.float32)
    iota_row = lax.broadcasted_iota(jnp.int32, (1, T), 1).astype(jnp.float32)
    rows = []
    for e in range(E):
        q01 = jnp.logical_and(slot == rank_f[:, e:e + 1],
                              wk[:, e:e + 1] > 0.0).astype(jnp.float32)
        rows.append(jnp.dot(iota_row, q01,
                            preferred_element_type=jnp.float32,
                            precision=lax.Precision.HIGHEST))
    sel_ref[...] = jnp.concatenate(rows, axis=0).astype(jnp.int32)


def _make_sc_gather(T, D, E, cap):
    B = E * cap
    NW = 32
    bpw = B // NW
    CH = 16
    nch = bpw // CH
    mesh = plsc.VectorSubcoreMesh(core_axis_name="c", subcore_axis_name="s")

    @functools.partial(
        pl.kernel, mesh=mesh,
        out_type=jax.ShapeDtypeStruct((B, D), jnp.float32),
        scratch_types=[
            pltpu.VMEM((CH,), jnp.int32),
            pltpu.VMEM((CH, D), jnp.float32),
            pltpu.SemaphoreType.DMA,
        ],
    )
    def gather(h_hbm, sel_hbm, out_hbm, idx_v, rows_v, sem):
        wid = lax.axis_index("s") * 2 + lax.axis_index("c")
        e = wid // (NW // E)
        within = (wid % (NW // E)) * bpw
        for c in range(nch):
            s0 = within + c * CH
            pltpu.sync_copy(sel_hbm.at[e, pl.ds(s0, CH)], idx_v)
            pltpu.async_copy(h_hbm.at[idx_v], rows_v, sem).wait()
            pltpu.sync_copy(rows_v, out_hbm.at[pl.ds(wid * bpw + c * CH, CH)])

    return gather


def _dense_body(cap, nf, xs_ref, wk_ref, rank_ref, wg_ref, wu_ref, wd_ref,
                out_ref, q_scr, xs_scr, ws_scr, ys_scr):
    e = pl.program_id(0)
    f = pl.program_id(1)
    T = wk_ref.shape[0]
    E = wk_ref.shape[1]

    @pl.when(jnp.logical_and(e == 0, f == 0))
    def _init():
        out_ref[...] = jnp.zeros_like(out_ref)

    @pl.when(f == 0)
    def _prep():
        oh = (lax.broadcasted_iota(jnp.int32, (E, 1), 0) == e).astype(
            jnp.float32)
        w_col = jnp.dot(wk_ref[...], oh, preferred_element_type=jnp.float32,
                        precision=lax.Precision.HIGHEST)
        rank_col = jnp.dot(rank_ref[...], oh,
                           preferred_element_type=jnp.float32,
                           precision=lax.Precision.HIGHEST)
        slot = lax.broadcasted_iota(jnp.int32, (T, cap), 1).astype(jnp.float32)
        q01f = jnp.logical_and(slot == rank_col,
                               w_col > 0.0).astype(jnp.float32)
        q_scr[...] = q01f.astype(jnp.bfloat16)
        ws_scr[...] = lax.dot_general(
            q01f, w_col, (((0,), (0,)), ((), ())),
            preferred_element_type=jnp.float32,
            precision=lax.Precision.HIGHEST)
        xs_scr[...] = xs_ref[...].astype(jnp.bfloat16)

    xs = xs_scr[...]
    wgb = wg_ref[0].astype(jnp.bfloat16)
    wub = wu_ref[0].astype(jnp.bfloat16)
    wdb = wd_ref[0].astype(jnp.bfloat16)
    g = jnp.dot(xs, wgb, preferred_element_type=jnp.float32)
    u = jnp.dot(xs, wub, preferred_element_type=jnp.float32)
    act = (g * (1.0 / (1.0 + jnp.exp(-g)))) * u
    contrib = jnp.dot(act.astype(jnp.bfloat16), wdb,
                      preferred_element_type=jnp.float32)

    @pl.when(f == 0)
    def _y0():
        ys_scr[...] = contrib

    @pl.when(f != 0)
    def _yacc():
        ys_scr[...] += contrib

    @pl.when(f == nf - 1)
    def _combine():
        ysw = (ys_scr[...] * ws_scr[...]).astype(jnp.bfloat16)
        out_ref[...] += jnp.dot(q_scr[...], ysw,
                                preferred_element_type=jnp.float32)


def kernel(hidden_states, w_gate, w_gate_proj, w_up_proj, w_down_proj):
    T, D = hidden_states.shape
    E = w_gate.shape[1]
    DFF = w_gate_proj.shape[2]
    CAP = int(math.ceil(T * _K / E * 1.25))
    NF = 2
    FBLK = DFF // NF

    ids, wk, rank, sel = pl.pallas_call(
        functools.partial(_routing_body, CAP),
        out_shape=(
            jax.ShapeDtypeStruct((T, _K), jnp.int32),
            jax.ShapeDtypeStruct((T, E), jnp.float32),
            jax.ShapeDtypeStruct((T, E), jnp.float32),
            jax.ShapeDtypeStruct((E, CAP), jnp.int32),
        ),
    )(hidden_states, w_gate)

    xs_all = _make_sc_gather(T, D, E, CAP)(hidden_states, sel)

    out = pl.pallas_call(
        functools.partial(_dense_body, CAP, NF),
        grid=(E, NF),
        in_specs=[
            pl.BlockSpec((CAP, D), lambda e, f: (e, 0)),
            pl.BlockSpec((T, E), lambda e, f: (0, 0)),
            pl.BlockSpec((T, E), lambda e, f: (0, 0)),
            pl.BlockSpec((1, D, FBLK), lambda e, f: (e, 0, f)),
            pl.BlockSpec((1, D, FBLK), lambda e, f: (e, 0, f)),
            pl.BlockSpec((1, FBLK, D), lambda e, f: (e, f, 0)),
        ],
        out_specs=pl.BlockSpec((T, D), lambda e, f: (0, 0)),
        out_shape=jax.ShapeDtypeStruct((T, D), jnp.float32),
        scratch_shapes=[
            pltpu.VMEM((T, CAP), jnp.bfloat16),
            pltpu.VMEM((CAP, D), jnp.bfloat16),
            pltpu.VMEM((CAP, 1), jnp.float32),
            pltpu.VMEM((CAP, D), jnp.float32),
        ],
    )(xs_all, wk, rank, w_gate_proj, w_up_proj, w_down_proj)

    return out, ids


# speedup vs baseline: 1.0024x; 1.0024x over previous
import functools
import math

import jax
import jax.numpy as jnp
from jax import lax
from jax.experimental import pallas as pl
from jax.experimental.pallas import tpu as pltpu
from jax.experimental.pallas import tpu_sc as plsc

_K = 2


def _cumsum_excl(x, T):
    acc = x
    k = 1
    while k < T:
        shifted = jnp.concatenate(
            [jnp.zeros((k,) + x.shape[1:], x.dtype), acc[:-k]], axis=0)
        acc = acc + shifted
        k *= 2
    return acc - x


def _routing_body(cap, h_ref, wg_ref, ids_ref, wk_ref, rank_ref, sel_ref):
    h = h_ref[...]
    wg = wg_ref[...]
    T = h.shape[0]
    E = wg.shape[1]
    logits = jnp.dot(h.astype(jnp.bfloat16), wg.astype(jnp.bfloat16),
                     preferred_element_type=jnp.float32)
    lane = lax.broadcasted_iota(jnp.int32, (T, E), 1)

    m = jnp.max(logits, axis=-1, keepdims=True)
    p = jnp.exp(logits - m)
    s = jnp.sum(p, axis=-1, keepdims=True)
    probs = p / s

    m1 = jnp.max(probs, axis=-1, keepdims=True)
    i1 = jnp.min(jnp.where(probs == m1, lane, E), axis=-1, keepdims=True)
    masked = jnp.where(lane == i1, -jnp.inf, probs)
    m2 = jnp.max(masked, axis=-1, keepdims=True)
    i2 = jnp.min(jnp.where(masked == m2, lane, E), axis=-1, keepdims=True)
    denom = m1 + m2
    w1 = m1 / denom
    w2 = m2 / denom
    ids_ref[...] = jnp.concatenate([i1, i2], axis=1)

    w_full = jnp.where(lane == i1, w1, 0.0) + jnp.where(lane == i2, w2, 0.0)

    keys = lax.bitcast_convert_type(w_full, jnp.int32)
    lo = jnp.zeros((1, E), jnp.int32)
    for b in range(30, -1, -1):
        trial = lo | (1 << b)
        cnt = jnp.sum((keys >= trial).astype(jnp.int32), axis=0, keepdims=True)
        lo = jnp.where(cnt >= cap, trial, lo)
    tau = lo
    g = jnp.sum((keys > tau).astype(jnp.int32), axis=0, keepdims=True)
    tie = (keys == tau)
    tie_rank = _cumsum_excl(tie.astype(jnp.int32), T)
    keep = (keys > tau) | (tie & (tie_rank < (cap - g)))
    wk = jnp.where(keep, w_full, 0.0)
    wk_ref[...] = wk

    pos = (wk > 0).astype(jnp.int32)
    rank = _cumsum_excl(pos, T)
    rank_f = rank.astype(jnp.float32)
    rank_ref[...] = rank_f

    slot = lax.broadcasted_iota(jnp.int32, (T, cap), 1).astype(jnp.float32)
    iota_row = lax.broadcasted_iota(jnp.int32, (1, T), 1).astype(jnp.float32)
    rows = []
    for e in range(E):
        q01 = jnp.logical_and(slot == rank_f[:, e:e + 1],
                              wk[:, e:e + 1] > 0.0).astype(jnp.float32)
        rows.append(jnp.dot(iota_row, q01,
                            preferred_element_type=jnp.float32,
                            precision=lax.Precision.HIGHEST))
    sel_ref[...] = jnp.concatenate(rows, axis=0).astype(jnp.int32)


def _make_sc_gather(T, D, E, cap):
    B = E * cap
    NW = 32
    bpw = B // NW
    CH = 16
    nch = bpw // CH
    mesh = plsc.VectorSubcoreMesh(core_axis_name="c", subcore_axis_name="s")

    @functools.partial(
        pl.kernel, mesh=mesh,
        out_type=jax.ShapeDtypeStruct((B, D), jnp.float32),
        scratch_types=[
            pltpu.VMEM((CH,), jnp.int32),
            pltpu.VMEM((CH,), jnp.int32),
            pltpu.VMEM((CH, D), jnp.float32),
            pltpu.VMEM((CH, D), jnp.float32),
            pltpu.SemaphoreType.DMA,
            pltpu.SemaphoreType.DMA,
        ],
    )
    def gather(h_hbm, sel_hbm, out_hbm, idx0, idx1, rows0, rows1, semg, semw):
        wid = lax.axis_index("s") * 2 + lax.axis_index("c")
        e = wid // (NW // E)
        within = (wid % (NW // E)) * bpw
        idxb = (idx0, idx1)
        bufs = (rows0, rows1)
        gd = [None, None]
        wr = [None, None]
        pltpu.sync_copy(sel_hbm.at[e, pl.ds(within, CH)], idx0)
        gd[0] = pltpu.async_copy(h_hbm.at[idx0], bufs[0], semg)
        for c in range(nch):
            b = c & 1
            gd[b].wait()
            if wr[1 - b] is not None:
                wr[1 - b].wait()
            if c + 1 < nch:
                pltpu.sync_copy(
                    sel_hbm.at[e, pl.ds(within + (c + 1) * CH, CH)],
                    idxb[1 - b])
                gd[1 - b] = pltpu.async_copy(h_hbm.at[idxb[1 - b]],
                                             bufs[1 - b], semg)
            wr[b] = pltpu.async_copy(
                bufs[b], out_hbm.at[pl.ds(wid * bpw + c * CH, CH)], semw)
        wr[(nch - 1) & 1].wait()

    return gather


def _dense_body(cap, nf, xs_ref, wk_ref, rank_ref, wg_ref, wu_ref, wd_ref,
                out_ref, q_scr, xs_scr, ws_scr, ys_scr):
    e = pl.program_id(0)
    f = pl.program_id(1)
    T = wk_ref.shape[0]
    E = wk_ref.shape[1]

    @pl.when(jnp.logical_and(e == 0, f == 0))
    def _init():
        out_ref[...] = jnp.zeros_like(out_ref)

    @pl.when(f == 0)
    def _prep():
        oh = (lax.broadcasted_iota(jnp.int32, (E, 1), 0) == e).astype(
            jnp.float32)
        w_col = jnp.dot(wk_ref[...], oh, preferred_element_type=jnp.float32,
                        precision=lax.Precision.HIGHEST)
        rank_col = jnp.dot(rank_ref[...], oh,
                           preferred_element_type=jnp.float32,
                           precision=lax.Precision.HIGHEST)
        slot = lax.broadcasted_iota(jnp.int32, (T, cap), 1).astype(jnp.float32)
        q01f = jnp.logical_and(slot == rank_col,
                               w_col > 0.0).astype(jnp.float32)
        q_scr[...] = q01f.astype(jnp.bfloat16)
        ws_scr[...] = lax.dot_general(
            q01f, w_col, (((0,), (0,)), ((), ())),
            preferred_element_type=jnp.float32,
            precision=lax.Precision.HIGHEST)
        xs_scr[...] = xs_ref[...].astype(jnp.bfloat16)

    xs = xs_scr[...]
    wgb = wg_ref[0].astype(jnp.bfloat16)
    wub = wu_ref[0].astype(jnp.bfloat16)
    wdb = wd_ref[0].astype(jnp.bfloat16)
    g = jnp.dot(xs, wgb, preferred_element_type=jnp.float32)
    u = jnp.dot(xs, wub, preferred_element_type=jnp.float32)
    act = (g * (1.0 / (1.0 + jnp.exp(-g)))) * u
    contrib = jnp.dot(act.astype(jnp.bfloat16), wdb,
                      preferred_element_type=jnp.float32)

    @pl.when(f == 0)
    def _y0():
        ys_scr[...] = contrib

    @pl.when(f != 0)
    def _yacc():
        ys_scr[...] += contrib

    @pl.when(f == nf - 1)
    def _combine():
        ysw = (ys_scr[...] * ws_scr[...]).astype(jnp.bfloat16)
        out_ref[...] += jnp.dot(q_scr[...], ysw,
                                preferred_element_type=jnp.float32)


def kernel(hidden_states, w_gate, w_gate_proj, w_up_proj, w_down_proj):
    T, D = hidden_states.shape
    E = w_gate.shape[1]
    DFF = w_gate_proj.shape[2]
    CAP = int(math.ceil(T * _K / E * 1.25))
    NF = 2
    FBLK = DFF // NF

    ids, wk, rank, sel = pl.pallas_call(
        functools.partial(_routing_body, CAP),
        out_shape=(
            jax.ShapeDtypeStruct((T, _K), jnp.int32),
            jax.ShapeDtypeStruct((T, E), jnp.float32),
            jax.ShapeDtypeStruct((T, E), jnp.float32),
            jax.ShapeDtypeStruct((E, CAP), jnp.int32),
        ),
    )(hidden_states, w_gate)

    xs_all = _make_sc_gather(T, D, E, CAP)(hidden_states, sel)

    out = pl.pallas_call(
        functools.partial(_dense_body, CAP, NF),
        grid=(E, NF),
        in_specs=[
            pl.BlockSpec((CAP, D), lambda e, f: (e, 0)),
            pl.BlockSpec((T, E), lambda e, f: (0, 0)),
            pl.BlockSpec((T, E), lambda e, f: (0, 0)),
            pl.BlockSpec((1, D, FBLK), lambda e, f: (e, 0, f)),
            pl.BlockSpec((1, D, FBLK), lambda e, f: (e, 0, f)),
            pl.BlockSpec((1, FBLK, D), lambda e, f: (e, f, 0)),
        ],
        out_specs=pl.BlockSpec((T, D), lambda e, f: (0, 0)),
        out_shape=jax.ShapeDtypeStruct((T, D), jnp.float32),
        scratch_shapes=[
            pltpu.VMEM((T, CAP), jnp.bfloat16),
            pltpu.VMEM((CAP, D), jnp.bfloat16),
            pltpu.VMEM((CAP, 1), jnp.float32),
            pltpu.VMEM((CAP, D), jnp.float32),
        ],
    )(xs_all, wk, rank, w_gate_proj, w_up_proj, w_down_proj)

    return out, ids
